# Initial kernel scaffold; baseline (speedup 1.0000x reference)
import jax
import jax.numpy as jnp
from jax.experimental import pallas as pl


def kernel(x, edge_index, batch, W_in, b_in, Wm, bm, Wu, bu, W_out, b_out):
    raise NotImplementedError("write your pallas kernel here")


# speedup vs baseline: 2.8859x; 2.8859x over previous
import functools

import jax
import jax.numpy as jnp
from jax import lax
from jax.experimental import pallas as pl
from jax.experimental.pallas import tpu as pltpu
from jax.experimental.pallas import tpu_sc as plsc

NC = 2
NS = 16
CHUNK = 128
G = 8

_HIGH = lax.Precision.HIGHEST


def _dot(a, b):
    return lax.dot_general(a, b, (((1,), (0,)), ((), ())), precision=_HIGH,
                           preferred_element_type=jnp.float32)


def _mlp_in(x, w_in, b_in, wm0, bm0, nb):
    n, d_in = x.shape
    ds = w_in.shape[1]
    h = ds // 2

    def body(x_ref, wi_ref, bi_ref, wm_ref, bm_ref, st_ref, msg_ref):
        s = jnp.maximum(_dot(x_ref[...], wi_ref[...]) + bi_ref[...], 0.0)
        st_ref[...] = s
        m = jnp.maximum(_dot(s, wm_ref[...]) + bm_ref[...], 0.0)
        msg_ref[0] = m[:, :h]
        msg_ref[1] = m[:, h:]

    return pl.pallas_call(
        body,
        grid=(n // nb,),
        in_specs=[
            pl.BlockSpec((nb, d_in), lambda i: (i, 0)),
            pl.BlockSpec((d_in, ds), lambda i: (0, 0)),
            pl.BlockSpec((1, ds), lambda i: (0, 0)),
            pl.BlockSpec((ds, ds), lambda i: (0, 0)),
            pl.BlockSpec((1, ds), lambda i: (0, 0)),
        ],
        out_specs=[
            pl.BlockSpec((nb, ds), lambda i: (i, 0)),
            pl.BlockSpec((2, nb, h), lambda i: (0, i, 0)),
        ],
        out_shape=[
            jax.ShapeDtypeStruct((n, ds), jnp.float32),
            jax.ShapeDtypeStruct((2, n, h), jnp.float32),
        ],
    )(x, w_in, b_in, wm0, bm0)


def _mlp_round(state, agg, wu, bu, wm, bm, nb):
    n, ds = state.shape
    h = ds // 2

    def body(st_ref, ag_ref, wu_ref, bu_ref, wm_ref, bm_ref, stn_ref, msg_ref):
        a = jnp.concatenate([ag_ref[0], ag_ref[1]], axis=1)
        u = jnp.maximum(_dot(a, wu_ref[...]) + bu_ref[...], 0.0)
        s = st_ref[...] + u
        stn_ref[...] = s
        m = jnp.maximum(_dot(s, wm_ref[...]) + bm_ref[...], 0.0)
        msg_ref[0] = m[:, :h]
        msg_ref[1] = m[:, h:]

    return pl.pallas_call(
        body,
        grid=(n // nb,),
        in_specs=[
            pl.BlockSpec((nb, ds), lambda i: (i, 0)),
            pl.BlockSpec((2, nb, h), lambda i: (0, i, 0)),
            pl.BlockSpec((ds, ds), lambda i: (0, 0)),
            pl.BlockSpec((1, ds), lambda i: (0, 0)),
            pl.BlockSpec((ds, ds), lambda i: (0, 0)),
            pl.BlockSpec((1, ds), lambda i: (0, 0)),
        ],
        out_specs=[
            pl.BlockSpec((nb, ds), lambda i: (i, 0)),
            pl.BlockSpec((2, nb, h), lambda i: (0, i, 0)),
        ],
        out_shape=[
            jax.ShapeDtypeStruct((n, ds), jnp.float32),
            jax.ShapeDtypeStruct((2, n, h), jnp.float32),
        ],
    )(state, agg, wu, bu, wm, bm)


def _mlp_out(state, agg, wu, bu, w_out, b_out, nb):
    n, ds = state.shape
    d_out = w_out.shape[1]

    def body(st_ref, ag_ref, wu_ref, bu_ref, wo_ref, bo_ref, o_ref):
        a = jnp.concatenate([ag_ref[0], ag_ref[1]], axis=1)
        u = jnp.maximum(_dot(a, wu_ref[...]) + bu_ref[...], 0.0)
        s = st_ref[...] + u
        o_ref[...] = _dot(s, wo_ref[...]) + bo_ref[...]

    return pl.pallas_call(
        body,
        grid=(n // nb,),
        in_specs=[
            pl.BlockSpec((nb, ds), lambda i: (i, 0)),
            pl.BlockSpec((2, nb, ds // 2), lambda i: (0, i, 0)),
            pl.BlockSpec((ds, ds), lambda i: (0, 0)),
            pl.BlockSpec((1, ds), lambda i: (0, 0)),
            pl.BlockSpec((ds, d_out), lambda i: (0, 0)),
            pl.BlockSpec((1, d_out), lambda i: (0, 0)),
        ],
        out_specs=pl.BlockSpec((nb, d_out), lambda i: (i, 0)),
        out_shape=jax.ShapeDtypeStruct((n, d_out), jnp.float32),
    )(state, agg, wu, bu, w_out, b_out)


def _sc_aggregate(msg2, src_t, dst_t, *, n, h, n_pad, ch):
    mesh = plsc.VectorSubcoreMesh(core_axis_name="c", subcore_axis_name="s")
    rpt = n_pad // NS
    ng = ch // G

    @functools.partial(
        pl.kernel,
        out_type=jax.ShapeDtypeStruct((2, n_pad, h), jnp.float32),
        mesh=mesh,
        scratch_types=[
            pltpu.VMEM((2, G, CHUNK), jnp.int32),
            pltpu.VMEM((2, G, CHUNK), jnp.int32),
            pltpu.VMEM((2, CHUNK, h), jnp.float32),
            pltpu.VMEM_SHARED((n_pad, h), jnp.float32),
            pltpu.SemaphoreType.DMA,
            pltpu.SemaphoreType.DMA,
            pltpu.SemaphoreType.DMA,
            pltpu.SemaphoreType.DMA,
        ],
    )
    def k(msg_hbm, src_hbm, dst_hbm, agg_hbm,
          srcg, dstg, rows_v, acc_sh, semA, semB, sem0, sem1):
        c = lax.axis_index("c")
        s = lax.axis_index("s")
        zvec = jnp.zeros((16,), jnp.float32)

        @pl.loop(0, CHUNK)
        def _(i):
            for j in range(h // 16):
                rows_v[0, i, pl.ds(j * 16, 16)] = zvec

        base = s * rpt
        for kb in range(rpt // CHUNK):
            pltpu.sync_copy(rows_v.at[0],
                            acc_sh.at[pl.ds(base + kb * CHUNK, CHUNK)])

        tbl = msg_hbm.at[c]
        rsem = (sem0, sem1)

        pltpu.async_copy(src_hbm.at[s, pl.ds(0, G)], srcg.at[0], semA)
        pltpu.async_copy(dst_hbm.at[s, pl.ds(0, G)], dstg.at[0], semA)
        pltpu.async_copy(src_hbm.at[s, pl.ds(G, G)], srcg.at[1], semB)
        pltpu.async_copy(dst_hbm.at[s, pl.ds(G, G)], dstg.at[1], semB)
        plsc.subcore_barrier()

        def do_group(g, a, sem):
            pltpu.make_async_copy(src_hbm.at[s, pl.ds(0, G)], srcg.at[a],
                                  sem).wait()
            pltpu.make_async_copy(dst_hbm.at[s, pl.ds(0, G)], dstg.at[a],
                                  sem).wait()
            pltpu.async_copy(tbl.at[srcg.at[a, 0]], rows_v.at[0], sem0)
            pltpu.async_copy(tbl.at[srcg.at[a, 1]], rows_v.at[1], sem1)
            for j in range(G):
                rb = j % 2
                pltpu.make_async_copy(tbl.at[srcg.at[a, j]], rows_v.at[rb],
                                      rsem[rb]).wait()
                pltpu.sync_copy(rows_v.at[rb], acc_sh.at[dstg.at[a, j]],
                                add=True)
                if j + 2 < G:
                    pltpu.async_copy(tbl.at[srcg.at[a, j + 2]], rows_v.at[rb],
                                     rsem[rb])

            @pl.when(g + 2 < ng)
            def _():
                pltpu.async_copy(src_hbm.at[s, pl.ds((g + 2) * G, G)],
                                 srcg.at[a], sem)
                pltpu.async_copy(dst_hbm.at[s, pl.ds((g + 2) * G, G)],
                                 dstg.at[a], sem)

        @pl.loop(0, ng, step=2)
        def _(g):
            do_group(g, 0, semA)
            do_group(g + 1, 1, semB)

        plsc.subcore_barrier()
        pltpu.sync_copy(acc_sh.at[pl.ds(base, rpt)],
                        agg_hbm.at[c, pl.ds(base, rpt)])

    return k(msg2, src_t, dst_t)


def kernel(x, edge_index, batch, W_in, b_in, Wm, bm, Wu, bu, W_out, b_out):
    n, _ = x.shape
    ds = W_in.shape[1]
    h = ds // 2
    e = edge_index.shape[1]
    rounds = Wm.shape[0]
    assert n % NS == 0 and ds % 32 == 0

    ch = -(-e // (NS * CHUNK))
    ch = -(-ch // (2 * G)) * (2 * G)
    ept = ch * CHUNK
    e_pad = ept * NS
    src = jnp.concatenate(
        [edge_index[0], jnp.zeros((e_pad - e,), jnp.int32)]).reshape(NS, ch, CHUNK)
    dst = jnp.concatenate(
        [edge_index[1], jnp.full((e_pad - e,), n, jnp.int32)]).reshape(NS, ch, CHUNK)

    n_pad = -(-(n + 1) // (NS * CHUNK)) * NS * CHUNK

    nb = 1000 if n % 1000 == 0 else n // NS

    b_in2 = b_in.reshape(1, ds)
    bm2 = bm.reshape(rounds, 1, ds)
    bu2 = bu.reshape(rounds, 1, ds)
    b_out2 = b_out.reshape(1, -1)

    state, msg = _mlp_in(x, W_in, b_in2, Wm[0], bm2[0], nb)
    for r in range(rounds):
        agg = _sc_aggregate(msg, src, dst, n=n, h=h, n_pad=n_pad, ch=ch)
        if r + 1 < rounds:
            state, msg = _mlp_round(state, agg, Wu[r], bu2[r], Wm[r + 1],
                                    bm2[r + 1], nb)
        else:
            out = _mlp_out(state, agg, Wu[r], bu2[r], W_out, b_out2, nb)
    return out


# speedup vs baseline: 2.9879x; 1.0353x over previous
import functools

import jax
import jax.numpy as jnp
from jax import lax
from jax.experimental import pallas as pl
from jax.experimental.pallas import tpu as pltpu
from jax.experimental.pallas import tpu_sc as plsc

NC = 2
NS = 16
CHUNK = 128
G = 8

_HIGH = lax.Precision.HIGHEST


def _dot(a, b):
    return lax.dot_general(a, b, (((1,), (0,)), ((), ())), precision=_HIGH,
                           preferred_element_type=jnp.float32)


def _mlp_in(x, w_in, b_in, wm0, bm0, nb):
    n, d_in = x.shape
    ds = w_in.shape[1]
    h = ds // 2

    def body(x_ref, wi_ref, bi_ref, wm_ref, bm_ref, st_ref, msg_ref):
        s = jnp.maximum(_dot(x_ref[...], wi_ref[...]) + bi_ref[...], 0.0)
        st_ref[...] = s
        m = jnp.maximum(_dot(s, wm_ref[...]) + bm_ref[...], 0.0)
        msg_ref[0] = m[:, :h]
        msg_ref[1] = m[:, h:]

    return pl.pallas_call(
        body,
        grid=(n // nb,),
        in_specs=[
            pl.BlockSpec((nb, d_in), lambda i: (i, 0)),
            pl.BlockSpec((d_in, ds), lambda i: (0, 0)),
            pl.BlockSpec((1, ds), lambda i: (0, 0)),
            pl.BlockSpec((ds, ds), lambda i: (0, 0)),
            pl.BlockSpec((1, ds), lambda i: (0, 0)),
        ],
        out_specs=[
            pl.BlockSpec((nb, ds), lambda i: (i, 0)),
            pl.BlockSpec((2, nb, h), lambda i: (0, i, 0)),
        ],
        out_shape=[
            jax.ShapeDtypeStruct((n, ds), jnp.float32),
            jax.ShapeDtypeStruct((2, n, h), jnp.float32),
        ],
    )(x, w_in, b_in, wm0, bm0)


def _mlp_round(state, agg, wu, bu, wm, bm, nb):
    n, ds = state.shape
    h = ds // 2

    def body(st_ref, ag_ref, wu_ref, bu_ref, wm_ref, bm_ref, stn_ref, msg_ref):
        a = jnp.concatenate([ag_ref[0], ag_ref[1]], axis=1)
        u = jnp.maximum(_dot(a, wu_ref[...]) + bu_ref[...], 0.0)
        s = st_ref[...] + u
        stn_ref[...] = s
        m = jnp.maximum(_dot(s, wm_ref[...]) + bm_ref[...], 0.0)
        msg_ref[0] = m[:, :h]
        msg_ref[1] = m[:, h:]

    return pl.pallas_call(
        body,
        grid=(n // nb,),
        in_specs=[
            pl.BlockSpec((nb, ds), lambda i: (i, 0)),
            pl.BlockSpec((2, nb, h), lambda i: (0, i, 0)),
            pl.BlockSpec((ds, ds), lambda i: (0, 0)),
            pl.BlockSpec((1, ds), lambda i: (0, 0)),
            pl.BlockSpec((ds, ds), lambda i: (0, 0)),
            pl.BlockSpec((1, ds), lambda i: (0, 0)),
        ],
        out_specs=[
            pl.BlockSpec((nb, ds), lambda i: (i, 0)),
            pl.BlockSpec((2, nb, h), lambda i: (0, i, 0)),
        ],
        out_shape=[
            jax.ShapeDtypeStruct((n, ds), jnp.float32),
            jax.ShapeDtypeStruct((2, n, h), jnp.float32),
        ],
    )(state, agg, wu, bu, wm, bm)


def _mlp_out(state, agg, wu, bu, w_out, b_out, nb):
    n, ds = state.shape
    d_out = w_out.shape[1]

    def body(st_ref, ag_ref, wu_ref, bu_ref, wo_ref, bo_ref, o_ref):
        a = jnp.concatenate([ag_ref[0], ag_ref[1]], axis=1)
        u = jnp.maximum(_dot(a, wu_ref[...]) + bu_ref[...], 0.0)
        s = st_ref[...] + u
        o_ref[...] = _dot(s, wo_ref[...]) + bo_ref[...]

    return pl.pallas_call(
        body,
        grid=(n // nb,),
        in_specs=[
            pl.BlockSpec((nb, ds), lambda i: (i, 0)),
            pl.BlockSpec((2, nb, ds // 2), lambda i: (0, i, 0)),
            pl.BlockSpec((ds, ds), lambda i: (0, 0)),
            pl.BlockSpec((1, ds), lambda i: (0, 0)),
            pl.BlockSpec((ds, d_out), lambda i: (0, 0)),
            pl.BlockSpec((1, d_out), lambda i: (0, 0)),
        ],
        out_specs=pl.BlockSpec((nb, d_out), lambda i: (i, 0)),
        out_shape=jax.ShapeDtypeStruct((n, d_out), jnp.float32),
    )(state, agg, wu, bu, w_out, b_out)


def _sc_aggregate(msg2, src_t, dst_t, *, n, h, n_pad, ch):
    mesh = plsc.VectorSubcoreMesh(core_axis_name="c", subcore_axis_name="s")
    rpt = n_pad // NS
    ng = ch // G

    @functools.partial(
        pl.kernel,
        out_type=jax.ShapeDtypeStruct((2, n_pad, h), jnp.float32),
        mesh=mesh,
        scratch_types=[
            pltpu.VMEM((2, G, CHUNK), jnp.int32),
            pltpu.VMEM((2, G, CHUNK), jnp.int32),
            pltpu.VMEM((2, CHUNK, h), jnp.float32),
            pltpu.VMEM_SHARED((n_pad, h), jnp.float32),
            pltpu.SemaphoreType.DMA,
            pltpu.SemaphoreType.DMA,
            pltpu.SemaphoreType.DMA,
            pltpu.SemaphoreType.DMA,
        ],
    )
    def k(msg_hbm, src_hbm, dst_hbm, agg_hbm,
          srcg, dstg, rows_v, acc_sh, semA, semB, sem0, sem1):
        c = lax.axis_index("c")
        s = lax.axis_index("s")
        zvec = jnp.zeros((16,), jnp.float32)

        @pl.loop(0, CHUNK)
        def _(i):
            for j in range(h // 16):
                rows_v[0, i, pl.ds(j * 16, 16)] = zvec

        base = s * rpt
        for kb in range(rpt // CHUNK):
            pltpu.sync_copy(rows_v.at[0],
                            acc_sh.at[pl.ds(base + kb * CHUNK, CHUNK)])

        tbl = msg_hbm.at[c]
        rsem = (sem0, sem1)
        isem = (semA, semB)

        def wait_idx(a):
            pltpu.make_async_copy(src_hbm.at[s, pl.ds(0, G)], srcg.at[a],
                                  isem[a]).wait()
            pltpu.make_async_copy(dst_hbm.at[s, pl.ds(0, G)], dstg.at[a],
                                  isem[a]).wait()

        def fetch_idx(goff, a):
            pltpu.async_copy(src_hbm.at[s, pl.ds(goff * G, G)], srcg.at[a],
                             isem[a])
            pltpu.async_copy(dst_hbm.at[s, pl.ds(goff * G, G)], dstg.at[a],
                             isem[a])

        fetch_idx(0, 0)
        fetch_idx(1, 1)
        plsc.subcore_barrier()
        wait_idx(0)
        pltpu.async_copy(tbl.at[srcg.at[0, 0]], rows_v.at[0], sem0)
        pltpu.async_copy(tbl.at[srcg.at[0, 1]], rows_v.at[1], sem1)

        def do_group(g, a):
            an = 1 - a
            for j in range(G):
                rb = j % 2
                pltpu.make_async_copy(tbl.at[srcg.at[a, j]], rows_v.at[rb],
                                      rsem[rb]).wait()
                pltpu.sync_copy(rows_v.at[rb], acc_sh.at[dstg.at[a, j]],
                                add=True)
                if j == G - 2:
                    @pl.when(g + 1 < ng)
                    def _():
                        wait_idx(an)
                if j + 2 < G:
                    pltpu.async_copy(tbl.at[srcg.at[a, j + 2]], rows_v.at[rb],
                                     rsem[rb])
                else:
                    @pl.when(g + 1 < ng)
                    def _():
                        pltpu.async_copy(tbl.at[srcg.at[an, j + 2 - G]],
                                         rows_v.at[rb], rsem[rb])

            @pl.when(g + 2 < ng)
            def _():
                fetch_idx(g + 2, a)

        @pl.loop(0, ng, step=2)
        def _(g):
            do_group(g, 0)
            do_group(g + 1, 1)

        plsc.subcore_barrier()
        pltpu.sync_copy(acc_sh.at[pl.ds(base, rpt)],
                        agg_hbm.at[c, pl.ds(base, rpt)])

    return k(msg2, src_t, dst_t)


def kernel(x, edge_index, batch, W_in, b_in, Wm, bm, Wu, bu, W_out, b_out):
    n, _ = x.shape
    ds = W_in.shape[1]
    h = ds // 2
    e = edge_index.shape[1]
    rounds = Wm.shape[0]
    assert n % NS == 0 and ds % 32 == 0

    ch = -(-e // (NS * CHUNK))
    ch = -(-ch // (2 * G)) * (2 * G)
    ept = ch * CHUNK
    e_pad = ept * NS
    src = jnp.concatenate(
        [edge_index[0], jnp.zeros((e_pad - e,), jnp.int32)]).reshape(NS, ch, CHUNK)
    dst = jnp.concatenate(
        [edge_index[1], jnp.full((e_pad - e,), n, jnp.int32)]).reshape(NS, ch, CHUNK)

    n_pad = -(-(n + 1) // (NS * CHUNK)) * NS * CHUNK

    nb = 1000 if n % 1000 == 0 else n // NS

    b_in2 = b_in.reshape(1, ds)
    bm2 = bm.reshape(rounds, 1, ds)
    bu2 = bu.reshape(rounds, 1, ds)
    b_out2 = b_out.reshape(1, -1)

    state, msg = _mlp_in(x, W_in, b_in2, Wm[0], bm2[0], nb)
    for r in range(rounds):
        agg = _sc_aggregate(msg, src, dst, n=n, h=h, n_pad=n_pad, ch=ch)
        if r + 1 < rounds:
            state, msg = _mlp_round(state, agg, Wu[r], bu2[r], Wm[r + 1],
                                    bm2[r + 1], nb)
        else:
            out = _mlp_out(state, agg, Wu[r], bu2[r], W_out, b_out2, nb)
    return out
